# baseline (speedup 1.0000x reference)
import functools

import jax
import jax.numpy as jnp
from jax import lax
from jax.experimental import pallas as pl
from jax.experimental.pallas import tpu as pltpu
from jax.experimental.pallas import tpu_sc as plsc

_VAR_W = 1.0
_DIST_W = 0.5
_M_VAR = 0.5
_M_DIST = 3.0

_L = 16
_NC = 2
_NS = 16
_B = 8
_N = 512 * 512
_BPC = _B // _NC
_WPB = _NS // _BPC
_NPW = _N // _WPB
_CH = 16384
_NCHUNK = _NPW // _CH
_UNROLL = 8


def _phase_kernel(feat_hbm, gt_hbm, table_hbm, xbuf, gbuf, sum_acc,
                  cnt_acc, pull_acc, means_v, stage, shared):
    c = lax.axis_index("c")
    s = lax.axis_index("s")
    b_local = s // _WPB
    q = s - b_local * _WPB
    base = (c * _BPC + b_local) * _N + q * _NPW

    lanes = lax.iota(jnp.int32, _L)
    ci = lanes
    ones = jnp.ones((_L,), jnp.float32)
    zeros = jnp.zeros((_L,), jnp.float32)

    for r in range(_L):
        sum_acc[pl.ds(r * _L, _L)] = zeros
        cnt_acc[pl.ds(r * _L, _L)] = zeros
        pull_acc[pl.ds(r * _L, _L)] = zeros

    def chunk1(t, carry):
        off = base + t * _CH
        pltpu.sync_copy(feat_hbm.at[pl.ds(off, _CH)], xbuf)
        pltpu.sync_copy(gt_hbm.at[pl.ds(off, _CH)], gbuf)

        @plsc.parallel_loop(0, _CH, _L, unroll=_UNROLL)
        def vec1(o):
            x = xbuf[pl.ds(o, _L)]
            g = gbuf[pl.ds(o, _L)]
            fi = g * _L + lanes
            plsc.addupdate_scatter(sum_acc, [fi], x)
            plsc.addupdate_scatter(cnt_acc, [fi], ones)

        return carry

    lax.fori_loop(0, _NCHUNK, chunk1, 0)

    sum_v = zeros
    cnt_v = zeros
    for cc in range(9):
        sel = ci == cc
        s_sc = jnp.sum(sum_acc[pl.ds(cc * _L, _L)])
        n_sc = jnp.sum(cnt_acc[pl.ds(cc * _L, _L)])
        sum_v = jnp.where(sel, zeros + s_sc, sum_v)
        cnt_v = jnp.where(sel, zeros + n_sc, cnt_v)

    stage[0] = sum_v
    stage[1] = cnt_v
    pltpu.sync_copy(stage.at[pl.ds(0, 2)], shared.at[s, pl.ds(0, 2)])
    plsc.subcore_barrier()

    g0 = b_local * _WPB
    tot_sum = zeros
    tot_cnt = zeros
    for j in range(_WPB):
        pltpu.sync_copy(shared.at[g0 + j, pl.ds(0, 2)], stage.at[pl.ds(2, 2)])
        tot_sum = tot_sum + stage[2]
        tot_cnt = tot_cnt + stage[3]
    mean_v = tot_sum / jnp.maximum(tot_cnt, 1.0)
    means_v[...] = mean_v

    def chunk2(t, carry):
        off = base + t * _CH
        pltpu.sync_copy(feat_hbm.at[pl.ds(off, _CH)], xbuf)
        pltpu.sync_copy(gt_hbm.at[pl.ds(off, _CH)], gbuf)

        @plsc.parallel_loop(0, _CH, _L, unroll=_UNROLL)
        def vec2(o):
            x = xbuf[pl.ds(o, _L)]
            g = gbuf[pl.ds(o, _L)]
            m = plsc.load_gather(means_v, [g])
            h = jnp.maximum(jnp.abs(x - m) - _M_VAR, 0.0)
            plsc.addupdate_scatter(pull_acc, [g * _L + lanes], h * h)

        return carry

    lax.fori_loop(0, _NCHUNK, chunk2, 0)

    pull_v = zeros
    for cc in range(9):
        pull_v = jnp.where(ci == cc,
                           zeros + jnp.sum(pull_acc[pl.ds(cc * _L, _L)]),
                           pull_v)
    stage[2] = pull_v
    pltpu.sync_copy(stage.at[pl.ds(2, 1)], shared.at[s, pl.ds(2, 1)])
    plsc.subcore_barrier()

    @pl.when(q == 0)
    def _():
        tot_pull = zeros
        for j in range(_WPB):
            pltpu.sync_copy(shared.at[g0 + j, pl.ds(2, 1)],
                            stage.at[pl.ds(3, 1)])
            tot_pull = tot_pull + stage[3]
        stage[0] = mean_v
        stage[1] = tot_cnt
        stage[2] = tot_pull / jnp.maximum(tot_cnt, 1.0)
        stage[3] = zeros
        pltpu.sync_copy(stage, table_hbm.at[c * _BPC + b_local])


def _final_kernel(table_hbm, out_hbm, tv, ov):
    c = lax.axis_index("c")
    s = lax.axis_index("s")

    @pl.when((c == 0) & (s == 0))
    def _():
        pltpu.sync_copy(table_hbm, tv)
        ci = lax.iota(jnp.int32, _L)
        cls_ok = (ci >= 1) & (ci <= 8)

        maxl = jnp.int32(0)
        for b in range(_B):
            cnt = tv[b, 1]
            maxl = jnp.maximum(maxl, jnp.max(jnp.where(cnt > 0, ci, 0)))

        pull_sum = jnp.float32(0.0)
        pull_cnt = jnp.float32(0.0)
        push_sum = jnp.float32(0.0)
        push_cnt = jnp.float32(0.0)
        for b in range(_B):
            mean = tv[b, 0]
            cnt = tv[b, 1]
            pterm = tv[b, 2]
            present = cls_ok & (cnt > 0.0) & (ci <= maxl)
            presf = jnp.where(present, 1.0, 0.0)
            pull_sum = pull_sum + jnp.sum(jnp.where(present, pterm, 0.0))
            pull_cnt = pull_cnt + jnp.sum(presf)
            for i in range(1, 9):
                sel = ci == i
                mi = jnp.sum(jnp.where(sel, mean, 0.0))
                pi = jnp.sum(jnp.where(sel, presf, 0.0))
                d = jnp.maximum(2.0 * _M_DIST - jnp.abs(mean - mi), 0.0)
                pairf = jnp.where(present & (~sel), pi, 0.0)
                push_sum = push_sum + jnp.sum(d * d * pairf)
                push_cnt = push_cnt + jnp.sum(pairf)

        zv = jnp.zeros((_L,), jnp.float32)
        pull_loss = (zv + pull_sum) / jnp.maximum(zv + pull_cnt, 1.0) * _VAR_W
        push_loss = (zv + push_sum) / jnp.maximum(zv + push_cnt, 1.0) * _DIST_W
        ov[...] = pull_loss + push_loss
        pltpu.sync_copy(ov, out_hbm)


def _build_calls():
    mesh = plsc.VectorSubcoreMesh(core_axis_name="c", subcore_axis_name="s",
                                  num_cores=_NC, num_subcores=_NS)
    params = pltpu.CompilerParams(needs_layout_passes=False)
    phase = pl.kernel(
        _phase_kernel,
        out_type=jax.ShapeDtypeStruct((_B, 4, _L), jnp.float32),
        mesh=mesh,
        compiler_params=params,
        scratch_types=[
            pltpu.VMEM((_CH,), jnp.float32),
            pltpu.VMEM((_CH,), jnp.int32),
            pltpu.VMEM((_L * _L,), jnp.float32),
            pltpu.VMEM((_L * _L,), jnp.float32),
            pltpu.VMEM((_L * _L,), jnp.float32),
            pltpu.VMEM((_L,), jnp.float32),
            pltpu.VMEM((4, _L), jnp.float32),
            pltpu.VMEM_SHARED((_NS, 4, _L), jnp.float32),
        ],
    )
    final = pl.kernel(
        _final_kernel,
        out_type=jax.ShapeDtypeStruct((_L,), jnp.float32),
        mesh=mesh,
        compiler_params=params,
        scratch_types=[
            pltpu.VMEM((_B, 4, _L), jnp.float32),
            pltpu.VMEM((_L,), jnp.float32),
        ],
    )
    return phase, final


def kernel(featmap, gt):
    phase, final = _build_calls()
    feat = featmap.reshape(-1)
    g = gt.reshape(-1).astype(jnp.int32)
    table = phase(feat, g)
    out = final(table)
    return out[0]


# speedup vs baseline: 1.0017x; 1.0017x over previous
import functools

import jax
import jax.numpy as jnp
from jax import lax
from jax.experimental import pallas as pl
from jax.experimental.pallas import tpu as pltpu
from jax.experimental.pallas import tpu_sc as plsc

_VAR_W = 1.0
_DIST_W = 0.5
_M_VAR = 0.5
_M_DIST = 3.0

_L = 16
_NC = 2
_NS = 16
_B = 8
_N = 512 * 512
_BPC = _B // _NC
_WPB = _NS // _BPC
_NPW = _N // _WPB
_CH = 16384
_NCHUNK = _NPW // _CH
_UNROLL = 16


def _phase_kernel(feat_hbm, gt_hbm, table_hbm, xbuf, gbuf, sum_acc,
                  cnt_acc, pull_acc, means_v, stage, shared):
    c = lax.axis_index("c")
    s = lax.axis_index("s")
    b_local = s // _WPB
    q = s - b_local * _WPB
    base = (c * _BPC + b_local) * _N + q * _NPW

    lanes = lax.iota(jnp.int32, _L)
    ci = lanes
    ones = jnp.ones((_L,), jnp.float32)
    zeros = jnp.zeros((_L,), jnp.float32)

    for r in range(_L):
        sum_acc[pl.ds(r * _L, _L)] = zeros
        cnt_acc[pl.ds(r * _L, _L)] = zeros
        pull_acc[pl.ds(r * _L, _L)] = zeros

    def chunk1(t, carry):
        off = base + t * _CH
        pltpu.sync_copy(feat_hbm.at[pl.ds(off, _CH)], xbuf)
        pltpu.sync_copy(gt_hbm.at[pl.ds(off, _CH)], gbuf)

        @plsc.parallel_loop(0, _CH, _L, unroll=_UNROLL)
        def vec1(o):
            x = xbuf[pl.ds(o, _L)]
            g = gbuf[pl.ds(o, _L)]
            fi = g * _L + lanes
            plsc.addupdate_scatter(sum_acc, [fi], x)
            plsc.addupdate_scatter(cnt_acc, [fi], ones)

        return carry

    lax.fori_loop(0, _NCHUNK, chunk1, 0)

    sum_v = zeros
    cnt_v = zeros
    for cc in range(9):
        sel = ci == cc
        s_sc = jnp.sum(sum_acc[pl.ds(cc * _L, _L)])
        n_sc = jnp.sum(cnt_acc[pl.ds(cc * _L, _L)])
        sum_v = jnp.where(sel, zeros + s_sc, sum_v)
        cnt_v = jnp.where(sel, zeros + n_sc, cnt_v)

    stage[0] = sum_v
    stage[1] = cnt_v
    pltpu.sync_copy(stage.at[pl.ds(0, 2)], shared.at[s, pl.ds(0, 2)])
    plsc.subcore_barrier()

    g0 = b_local * _WPB
    tot_sum = zeros
    tot_cnt = zeros
    for j in range(_WPB):
        pltpu.sync_copy(shared.at[g0 + j, pl.ds(0, 2)], stage.at[pl.ds(2, 2)])
        tot_sum = tot_sum + stage[2]
        tot_cnt = tot_cnt + stage[3]
    mean_v = tot_sum / jnp.maximum(tot_cnt, 1.0)
    means_v[...] = mean_v

    def chunk2(t, carry):
        off = base + t * _CH
        pltpu.sync_copy(feat_hbm.at[pl.ds(off, _CH)], xbuf)
        pltpu.sync_copy(gt_hbm.at[pl.ds(off, _CH)], gbuf)

        @plsc.parallel_loop(0, _CH, _L, unroll=_UNROLL)
        def vec2(o):
            x = xbuf[pl.ds(o, _L)]
            g = gbuf[pl.ds(o, _L)]
            m = plsc.load_gather(means_v, [g])
            h = jnp.maximum(jnp.abs(x - m) - _M_VAR, 0.0)
            plsc.addupdate_scatter(pull_acc, [g * _L + lanes], h * h)

        return carry

    lax.fori_loop(0, _NCHUNK, chunk2, 0)

    pull_v = zeros
    for cc in range(9):
        pull_v = jnp.where(ci == cc,
                           zeros + jnp.sum(pull_acc[pl.ds(cc * _L, _L)]),
                           pull_v)
    stage[2] = pull_v
    pltpu.sync_copy(stage.at[pl.ds(2, 1)], shared.at[s, pl.ds(2, 1)])
    plsc.subcore_barrier()

    @pl.when(q == 0)
    def _():
        tot_pull = zeros
        for j in range(_WPB):
            pltpu.sync_copy(shared.at[g0 + j, pl.ds(2, 1)],
                            stage.at[pl.ds(3, 1)])
            tot_pull = tot_pull + stage[3]
        stage[0] = mean_v
        stage[1] = tot_cnt
        stage[2] = tot_pull / jnp.maximum(tot_cnt, 1.0)
        stage[3] = zeros
        pltpu.sync_copy(stage, table_hbm.at[c * _BPC + b_local])


def _final_kernel(table_hbm, out_hbm, tv, ov):
    c = lax.axis_index("c")
    s = lax.axis_index("s")

    @pl.when((c == 0) & (s == 0))
    def _():
        pltpu.sync_copy(table_hbm, tv)
        ci = lax.iota(jnp.int32, _L)
        cls_ok = (ci >= 1) & (ci <= 8)

        maxl = jnp.int32(0)
        for b in range(_B):
            cnt = tv[b, 1]
            maxl = jnp.maximum(maxl, jnp.max(jnp.where(cnt > 0, ci, 0)))

        pull_sum = jnp.float32(0.0)
        pull_cnt = jnp.float32(0.0)
        push_sum = jnp.float32(0.0)
        push_cnt = jnp.float32(0.0)
        for b in range(_B):
            mean = tv[b, 0]
            cnt = tv[b, 1]
            pterm = tv[b, 2]
            present = cls_ok & (cnt > 0.0) & (ci <= maxl)
            presf = jnp.where(present, 1.0, 0.0)
            pull_sum = pull_sum + jnp.sum(jnp.where(present, pterm, 0.0))
            pull_cnt = pull_cnt + jnp.sum(presf)
            for i in range(1, 9):
                sel = ci == i
                mi = jnp.sum(jnp.where(sel, mean, 0.0))
                pi = jnp.sum(jnp.where(sel, presf, 0.0))
                d = jnp.maximum(2.0 * _M_DIST - jnp.abs(mean - mi), 0.0)
                pairf = jnp.where(present & (~sel), pi, 0.0)
                push_sum = push_sum + jnp.sum(d * d * pairf)
                push_cnt = push_cnt + jnp.sum(pairf)

        zv = jnp.zeros((_L,), jnp.float32)
        pull_loss = (zv + pull_sum) / jnp.maximum(zv + pull_cnt, 1.0) * _VAR_W
        push_loss = (zv + push_sum) / jnp.maximum(zv + push_cnt, 1.0) * _DIST_W
        ov[...] = pull_loss + push_loss
        pltpu.sync_copy(ov, out_hbm)


def _build_calls():
    mesh = plsc.VectorSubcoreMesh(core_axis_name="c", subcore_axis_name="s",
                                  num_cores=_NC, num_subcores=_NS)
    params = pltpu.CompilerParams(needs_layout_passes=False)
    phase = pl.kernel(
        _phase_kernel,
        out_type=jax.ShapeDtypeStruct((_B, 4, _L), jnp.float32),
        mesh=mesh,
        compiler_params=params,
        scratch_types=[
            pltpu.VMEM((_CH,), jnp.float32),
            pltpu.VMEM((_CH,), jnp.int32),
            pltpu.VMEM((_L * _L,), jnp.float32),
            pltpu.VMEM((_L * _L,), jnp.float32),
            pltpu.VMEM((_L * _L,), jnp.float32),
            pltpu.VMEM((_L,), jnp.float32),
            pltpu.VMEM((4, _L), jnp.float32),
            pltpu.VMEM_SHARED((_NS, 4, _L), jnp.float32),
        ],
    )
    final = pl.kernel(
        _final_kernel,
        out_type=jax.ShapeDtypeStruct((_L,), jnp.float32),
        mesh=mesh,
        compiler_params=params,
        scratch_types=[
            pltpu.VMEM((_B, 4, _L), jnp.float32),
            pltpu.VMEM((_L,), jnp.float32),
        ],
    )
    return phase, final


def kernel(featmap, gt):
    phase, final = _build_calls()
    feat = featmap.reshape(-1)
    g = gt.reshape(-1).astype(jnp.int32)
    table = phase(feat, g)
    out = final(table)
    return out[0]


# speedup vs baseline: 1.1222x; 1.1203x over previous
import functools

import jax
import jax.numpy as jnp
from jax import lax
from jax.experimental import pallas as pl
from jax.experimental.pallas import tpu as pltpu
from jax.experimental.pallas import tpu_sc as plsc

_VAR_W = 1.0
_DIST_W = 0.5
_M_VAR = 0.5
_M_DIST = 3.0

_L = 16
_NC = 2
_NS = 16
_B = 8
_N = 512 * 512
_BPC = _B // _NC
_WPB = _NS // _BPC
_NPW = _N // _WPB
_CH = 16384
_NCHUNK = _NPW // _CH
_UNROLL = 8


def _phase_kernel(feat_hbm, gt_hbm, table_hbm, xbuf, gbuf, sum_acc,
                  cnt_acc, pull_acc, means_v, stage, shared,
                  sx0, sx1, sg0, sg1):
    c = lax.axis_index("c")
    s = lax.axis_index("s")
    b_local = s // _WPB
    q = s - b_local * _WPB
    base = (c * _BPC + b_local) * _N + q * _NPW
    sx = (sx0, sx1)
    sg = (sg0, sg1)

    def start_copy(t):
        bi = t & 1
        off = base + t * _CH
        dx = pltpu.async_copy(feat_hbm.at[pl.ds(off, _CH)], xbuf.at[bi], sx[bi])
        dg = pltpu.async_copy(gt_hbm.at[pl.ds(off, _CH)], gbuf.at[bi], sg[bi])
        return dx, dg

    lanes = lax.iota(jnp.int32, _L)
    ci = lanes
    ones = jnp.ones((_L,), jnp.float32)
    zeros = jnp.zeros((_L,), jnp.float32)

    pend = [None, None]
    pend[0] = start_copy(0)

    for r in range(_L):
        sum_acc[pl.ds(r * _L, _L)] = zeros
        cnt_acc[pl.ds(r * _L, _L)] = zeros
        pull_acc[pl.ds(r * _L, _L)] = zeros

    for t in range(_NCHUNK):
        bi = t & 1
        if t + 1 < _NCHUNK:
            pend[(t + 1) & 1] = start_copy(t + 1)
        dx, dg = pend[bi]
        dx.wait()
        dg.wait()

        @plsc.parallel_loop(0, _CH, _L, unroll=_UNROLL)
        def vec1(o):
            x = xbuf[bi, pl.ds(o, _L)]
            g = gbuf[bi, pl.ds(o, _L)]
            fi = g * _L + lanes
            plsc.addupdate_scatter(sum_acc, [fi], x)
            plsc.addupdate_scatter(cnt_acc, [fi], ones)

    pend[0] = start_copy(0)

    sum_v = zeros
    cnt_v = zeros
    for cc in range(9):
        sel = ci == cc
        s_sc = jnp.sum(sum_acc[pl.ds(cc * _L, _L)])
        n_sc = jnp.sum(cnt_acc[pl.ds(cc * _L, _L)])
        sum_v = jnp.where(sel, zeros + s_sc, sum_v)
        cnt_v = jnp.where(sel, zeros + n_sc, cnt_v)

    stage[0] = sum_v
    stage[1] = cnt_v
    pltpu.sync_copy(stage.at[pl.ds(0, 2)], shared.at[s, pl.ds(0, 2)])
    plsc.subcore_barrier()

    g0 = b_local * _WPB
    tot_sum = zeros
    tot_cnt = zeros
    for j in range(_WPB):
        pltpu.sync_copy(shared.at[g0 + j, pl.ds(0, 2)], stage.at[pl.ds(2, 2)])
        tot_sum = tot_sum + stage[2]
        tot_cnt = tot_cnt + stage[3]
    mean_v = tot_sum / jnp.maximum(tot_cnt, 1.0)
    means_v[...] = mean_v

    for t in range(_NCHUNK):
        bi = t & 1
        if t + 1 < _NCHUNK:
            pend[(t + 1) & 1] = start_copy(t + 1)
        dx, dg = pend[bi]
        dx.wait()
        dg.wait()

        @plsc.parallel_loop(0, _CH, _L, unroll=_UNROLL)
        def vec2(o):
            x = xbuf[bi, pl.ds(o, _L)]
            g = gbuf[bi, pl.ds(o, _L)]
            m = plsc.load_gather(means_v, [g])
            h = jnp.maximum(jnp.abs(x - m) - _M_VAR, 0.0)
            plsc.addupdate_scatter(pull_acc, [g * _L + lanes], h * h)

    pull_v = zeros
    for cc in range(9):
        pull_v = jnp.where(ci == cc,
                           zeros + jnp.sum(pull_acc[pl.ds(cc * _L, _L)]),
                           pull_v)
    stage[2] = pull_v
    pltpu.sync_copy(stage.at[pl.ds(2, 1)], shared.at[s, pl.ds(2, 1)])
    plsc.subcore_barrier()

    @pl.when(q == 0)
    def _():
        tot_pull = zeros
        for j in range(_WPB):
            pltpu.sync_copy(shared.at[g0 + j, pl.ds(2, 1)],
                            stage.at[pl.ds(3, 1)])
            tot_pull = tot_pull + stage[3]
        stage[0] = mean_v
        stage[1] = tot_cnt
        stage[2] = tot_pull / jnp.maximum(tot_cnt, 1.0)
        stage[3] = zeros
        pltpu.sync_copy(stage, table_hbm.at[c * _BPC + b_local])


def _final_kernel(table_hbm, out_hbm, tv, ov):
    c = lax.axis_index("c")
    s = lax.axis_index("s")

    @pl.when((c == 0) & (s == 0))
    def _():
        pltpu.sync_copy(table_hbm, tv)
        ci = lax.iota(jnp.int32, _L)
        cls_ok = (ci >= 1) & (ci <= 8)

        maxl = jnp.int32(0)
        for b in range(_B):
            cnt = tv[b, 1]
            maxl = jnp.maximum(maxl, jnp.max(jnp.where(cnt > 0, ci, 0)))

        pull_sum = jnp.float32(0.0)
        pull_cnt = jnp.float32(0.0)
        push_sum = jnp.float32(0.0)
        push_cnt = jnp.float32(0.0)
        for b in range(_B):
            mean = tv[b, 0]
            cnt = tv[b, 1]
            pterm = tv[b, 2]
            present = cls_ok & (cnt > 0.0) & (ci <= maxl)
            presf = jnp.where(present, 1.0, 0.0)
            pull_sum = pull_sum + jnp.sum(jnp.where(present, pterm, 0.0))
            pull_cnt = pull_cnt + jnp.sum(presf)
            for i in range(1, 9):
                sel = ci == i
                mi = jnp.sum(jnp.where(sel, mean, 0.0))
                pi = jnp.sum(jnp.where(sel, presf, 0.0))
                d = jnp.maximum(2.0 * _M_DIST - jnp.abs(mean - mi), 0.0)
                pairf = jnp.where(present & (~sel), pi, 0.0)
                push_sum = push_sum + jnp.sum(d * d * pairf)
                push_cnt = push_cnt + jnp.sum(pairf)

        zv = jnp.zeros((_L,), jnp.float32)
        pull_loss = (zv + pull_sum) / jnp.maximum(zv + pull_cnt, 1.0) * _VAR_W
        push_loss = (zv + push_sum) / jnp.maximum(zv + push_cnt, 1.0) * _DIST_W
        ov[...] = pull_loss + push_loss
        pltpu.sync_copy(ov, out_hbm)


def _build_calls():
    mesh = plsc.VectorSubcoreMesh(core_axis_name="c", subcore_axis_name="s",
                                  num_cores=_NC, num_subcores=_NS)
    params = pltpu.CompilerParams(needs_layout_passes=False)
    phase = pl.kernel(
        _phase_kernel,
        out_type=jax.ShapeDtypeStruct((_B, 4, _L), jnp.float32),
        mesh=mesh,
        compiler_params=params,
        scratch_types=[
            pltpu.VMEM((2, _CH), jnp.float32),
            pltpu.VMEM((2, _CH), jnp.int32),
            pltpu.VMEM((_L * _L,), jnp.float32),
            pltpu.VMEM((_L * _L,), jnp.float32),
            pltpu.VMEM((_L * _L,), jnp.float32),
            pltpu.VMEM((_L,), jnp.float32),
            pltpu.VMEM((4, _L), jnp.float32),
            pltpu.VMEM_SHARED((_NS, 4, _L), jnp.float32),
            pltpu.SemaphoreType.DMA,
            pltpu.SemaphoreType.DMA,
            pltpu.SemaphoreType.DMA,
            pltpu.SemaphoreType.DMA,
        ],
    )
    final = pl.kernel(
        _final_kernel,
        out_type=jax.ShapeDtypeStruct((_L,), jnp.float32),
        mesh=mesh,
        compiler_params=params,
        scratch_types=[
            pltpu.VMEM((_B, 4, _L), jnp.float32),
            pltpu.VMEM((_L,), jnp.float32),
        ],
    )
    return phase, final


def kernel(featmap, gt):
    phase, final = _build_calls()
    feat = featmap.reshape(-1)
    g = gt.reshape(-1).astype(jnp.int32)
    table = phase(feat, g)
    out = final(table)
    return out[0]


# speedup vs baseline: 1.2797x; 1.1404x over previous
import functools

import jax
import jax.numpy as jnp
from jax import lax
from jax.experimental import pallas as pl
from jax.experimental.pallas import tpu as pltpu
from jax.experimental.pallas import tpu_sc as plsc

_VAR_W = 1.0
_DIST_W = 0.5
_M_VAR = 0.5
_M_DIST = 3.0

_L = 16
_NC = 2
_NS = 16
_B = 8
_N = 512 * 512
_BPC = _B // _NC
_WPB = _NS // _BPC
_NPW = _N // _WPB
_ROWS = 32
_CH = _ROWS * 512
_NCHUNK = _NPW // _CH


def _phase_kernel(feat_hbm, gt_hbm, table_hbm, xbuf, gbuf, sum_acc,
                  cnt_acc, pull_acc, means_v, stage, shared,
                  sx0, sx1, sg0, sg1):
    c = lax.axis_index("c")
    s = lax.axis_index("s")
    b_local = s // _WPB
    q = s - b_local * _WPB
    b_glob = c * _BPC + b_local
    row0 = q * (512 // _WPB)
    sx = (sx0, sx1)
    sg = (sg0, sg1)

    def start_copy(t):
        bi = t & 1
        r0 = row0 + t * _ROWS
        dx = pltpu.async_copy(feat_hbm.at[b_glob, pl.ds(r0, _ROWS), :],
                              xbuf.at[bi], sx[bi])
        dg = pltpu.async_copy(gt_hbm.at[b_glob, pl.ds(r0, _ROWS), :],
                              gbuf.at[bi], sg[bi])
        return dx, dg

    lanes = lax.iota(jnp.int32, _L)
    ci = lanes
    ones = jnp.ones((_L,), jnp.float32)
    zeros = jnp.zeros((_L,), jnp.float32)

    pend = [None, None]
    pend[0] = start_copy(0)

    for r in range(_L):
        sum_acc[pl.ds(r * _L, _L)] = zeros
        cnt_acc[pl.ds(r * _L, _L)] = zeros
        pull_acc[pl.ds(r * _L, _L)] = zeros

    for t in range(_NCHUNK):
        bi = t & 1
        if t + 1 < _NCHUNK:
            pend[(t + 1) & 1] = start_copy(t + 1)
        dx, dg = pend[bi]
        dx.wait()
        dg.wait()

        @plsc.parallel_loop(0, _ROWS, 1)
        def vec1(r):
            for u in range(512 // _L):
                x = xbuf[bi, r, pl.ds(u * _L, _L)]
                g = gbuf[bi, r, pl.ds(u * _L, _L)]
                fi = g * _L + lanes
                plsc.addupdate_scatter(sum_acc, [fi], x)
                plsc.addupdate_scatter(cnt_acc, [fi], ones)

    pend[0] = start_copy(0)

    sum_v = zeros
    cnt_v = zeros
    for cc in range(9):
        sel = ci == cc
        s_sc = jnp.sum(sum_acc[pl.ds(cc * _L, _L)])
        n_sc = jnp.sum(cnt_acc[pl.ds(cc * _L, _L)])
        sum_v = jnp.where(sel, zeros + s_sc, sum_v)
        cnt_v = jnp.where(sel, zeros + n_sc, cnt_v)

    stage[0] = sum_v
    stage[1] = cnt_v
    pltpu.sync_copy(stage.at[pl.ds(0, 2)], shared.at[s, pl.ds(0, 2)])
    plsc.subcore_barrier()

    g0 = b_local * _WPB
    tot_sum = zeros
    tot_cnt = zeros
    for j in range(_WPB):
        pltpu.sync_copy(shared.at[g0 + j, pl.ds(0, 2)], stage.at[pl.ds(2, 2)])
        tot_sum = tot_sum + stage[2]
        tot_cnt = tot_cnt + stage[3]
    mean_v = tot_sum / jnp.maximum(tot_cnt, 1.0)
    means_v[...] = mean_v

    for t in range(_NCHUNK):
        bi = t & 1
        if t + 1 < _NCHUNK:
            pend[(t + 1) & 1] = start_copy(t + 1)
        dx, dg = pend[bi]
        dx.wait()
        dg.wait()

        @plsc.parallel_loop(0, _ROWS, 1)
        def vec2(r):
            for u in range(512 // _L):
                x = xbuf[bi, r, pl.ds(u * _L, _L)]
                g = gbuf[bi, r, pl.ds(u * _L, _L)]
                m = plsc.load_gather(means_v, [g])
                h = jnp.maximum(jnp.abs(x - m) - _M_VAR, 0.0)
                plsc.addupdate_scatter(pull_acc, [g * _L + lanes], h * h)

    pull_v = zeros
    for cc in range(9):
        pull_v = jnp.where(ci == cc,
                           zeros + jnp.sum(pull_acc[pl.ds(cc * _L, _L)]),
                           pull_v)
    stage[2] = pull_v
    pltpu.sync_copy(stage.at[pl.ds(2, 1)], shared.at[s, pl.ds(2, 1)])
    plsc.subcore_barrier()

    @pl.when(q == 0)
    def _():
        tot_pull = zeros
        for j in range(_WPB):
            pltpu.sync_copy(shared.at[g0 + j, pl.ds(2, 1)],
                            stage.at[pl.ds(3, 1)])
            tot_pull = tot_pull + stage[3]
        stage[0] = mean_v
        stage[1] = tot_cnt
        stage[2] = tot_pull / jnp.maximum(tot_cnt, 1.0)
        stage[3] = zeros
        pltpu.sync_copy(stage, table_hbm.at[c * _BPC + b_local])


def _final_kernel(table_hbm, out_hbm, tv, ov):
    c = lax.axis_index("c")
    s = lax.axis_index("s")

    @pl.when((c == 0) & (s == 0))
    def _():
        pltpu.sync_copy(table_hbm, tv)
        ci = lax.iota(jnp.int32, _L)
        cls_ok = (ci >= 1) & (ci <= 8)

        maxl = jnp.int32(0)
        for b in range(_B):
            cnt = tv[b, 1]
            maxl = jnp.maximum(maxl, jnp.max(jnp.where(cnt > 0, ci, 0)))

        pull_sum = jnp.float32(0.0)
        pull_cnt = jnp.float32(0.0)
        push_sum = jnp.float32(0.0)
        push_cnt = jnp.float32(0.0)
        for b in range(_B):
            mean = tv[b, 0]
            cnt = tv[b, 1]
            pterm = tv[b, 2]
            present = cls_ok & (cnt > 0.0) & (ci <= maxl)
            presf = jnp.where(present, 1.0, 0.0)
            pull_sum = pull_sum + jnp.sum(jnp.where(present, pterm, 0.0))
            pull_cnt = pull_cnt + jnp.sum(presf)
            for i in range(1, 9):
                sel = ci == i
                mi = jnp.sum(jnp.where(sel, mean, 0.0))
                pi = jnp.sum(jnp.where(sel, presf, 0.0))
                d = jnp.maximum(2.0 * _M_DIST - jnp.abs(mean - mi), 0.0)
                pairf = jnp.where(present & (~sel), pi, 0.0)
                push_sum = push_sum + jnp.sum(d * d * pairf)
                push_cnt = push_cnt + jnp.sum(pairf)

        zv = jnp.zeros((_L,), jnp.float32)
        pull_loss = (zv + pull_sum) / jnp.maximum(zv + pull_cnt, 1.0) * _VAR_W
        push_loss = (zv + push_sum) / jnp.maximum(zv + push_cnt, 1.0) * _DIST_W
        ov[...] = pull_loss + push_loss
        pltpu.sync_copy(ov, out_hbm)


def _build_calls():
    mesh = plsc.VectorSubcoreMesh(core_axis_name="c", subcore_axis_name="s",
                                  num_cores=_NC, num_subcores=_NS)
    params = pltpu.CompilerParams(needs_layout_passes=False)
    phase_params = pltpu.CompilerParams(needs_layout_passes=False,
                                        use_tc_tiling_on_sc=True)
    phase = pl.kernel(
        _phase_kernel,
        out_type=jax.ShapeDtypeStruct((_B, 4, _L), jnp.float32),
        mesh=mesh,
        compiler_params=phase_params,
        scratch_types=[
            pltpu.VMEM((2, _ROWS, 512), jnp.float32),
            pltpu.VMEM((2, _ROWS, 512), jnp.int32),
            pltpu.VMEM((_L * _L,), jnp.float32),
            pltpu.VMEM((_L * _L,), jnp.float32),
            pltpu.VMEM((_L * _L,), jnp.float32),
            pltpu.VMEM((_L,), jnp.float32),
            pltpu.VMEM((4, _L), jnp.float32),
            pltpu.VMEM_SHARED((_NS, 4, _L), jnp.float32),
            pltpu.SemaphoreType.DMA,
            pltpu.SemaphoreType.DMA,
            pltpu.SemaphoreType.DMA,
            pltpu.SemaphoreType.DMA,
        ],
    )
    final = pl.kernel(
        _final_kernel,
        out_type=jax.ShapeDtypeStruct((_L,), jnp.float32),
        mesh=mesh,
        compiler_params=params,
        scratch_types=[
            pltpu.VMEM((_B, 4, _L), jnp.float32),
            pltpu.VMEM((_L,), jnp.float32),
        ],
    )
    return phase, final


def kernel(featmap, gt):
    phase, final = _build_calls()
    feat = featmap.reshape(_B, 512, 512)
    g = gt.reshape(_B, 512, 512).astype(jnp.int32)
    table = phase(feat, g)
    out = final(table)
    return out[0]


# speedup vs baseline: 1.2833x; 1.0028x over previous
import functools

import jax
import jax.numpy as jnp
from jax import lax
from jax.experimental import pallas as pl
from jax.experimental.pallas import tpu as pltpu
from jax.experimental.pallas import tpu_sc as plsc

_VAR_W = 1.0
_DIST_W = 0.5
_M_VAR = 0.5
_M_DIST = 3.0

_L = 16
_NC = 2
_NS = 16
_B = 8
_N = 512 * 512
_BPC = _B // _NC
_WPB = _NS // _BPC
_NPW = _N // _WPB
_ROWS = 32
_CH = _ROWS * 512
_NCHUNK = _NPW // _CH


def _phase_kernel(feat_hbm, gt_hbm, table_hbm, xbuf, gbuf, sum_acc,
                  cnt_acc, pull_acc, means_v, stage, shared,
                  sx0, sx1, sg0, sg1):
    c = lax.axis_index("c")
    s = lax.axis_index("s")
    b_local = s // _WPB
    q = s - b_local * _WPB
    b_glob = c * _BPC + b_local
    row0 = q * (512 // _WPB)
    sx = (sx0, sx1)
    sg = (sg0, sg1)

    def start_copy(t):
        bi = t & 1
        r0 = row0 + t * _ROWS
        dx = pltpu.async_copy(feat_hbm.at[b_glob, pl.ds(r0, _ROWS), :],
                              xbuf.at[bi], sx[bi])
        dg = pltpu.async_copy(gt_hbm.at[b_glob, pl.ds(r0, _ROWS), :],
                              gbuf.at[bi], sg[bi])
        return dx, dg

    lanes = lax.iota(jnp.int32, _L)
    ci = lanes
    ones = jnp.ones((_L,), jnp.float32)
    zeros = jnp.zeros((_L,), jnp.float32)

    pend = [None, None]
    pend[0] = start_copy(0)

    for r in range(2 * _L):
        sum_acc[pl.ds(r * _L, _L)] = zeros
        cnt_acc[pl.ds(r * _L, _L)] = zeros
        pull_acc[pl.ds(r * _L, _L)] = zeros

    for t in range(_NCHUNK):
        bi = t & 1
        if t + 1 < _NCHUNK:
            pend[(t + 1) & 1] = start_copy(t + 1)
        dx, dg = pend[bi]
        dx.wait()
        dg.wait()

        @plsc.parallel_loop(0, _ROWS, 1)
        def vec1(r):
            for u in range(512 // _L):
                x = xbuf[bi, r, pl.ds(u * _L, _L)]
                g = gbuf[bi, r, pl.ds(u * _L, _L)]
                fi = g * _L + lanes + (u & 1) * (_L * _L)
                plsc.addupdate_scatter(sum_acc, [fi], x)
                plsc.addupdate_scatter(cnt_acc, [fi], ones)

    pend[0] = start_copy(0)

    sum_v = zeros
    cnt_v = zeros
    for cc in range(9):
        sel = ci == cc
        s_sc = jnp.sum(sum_acc[pl.ds(cc * _L, _L)] +
                       sum_acc[pl.ds(_L * _L + cc * _L, _L)])
        n_sc = jnp.sum(cnt_acc[pl.ds(cc * _L, _L)] +
                       cnt_acc[pl.ds(_L * _L + cc * _L, _L)])
        sum_v = jnp.where(sel, zeros + s_sc, sum_v)
        cnt_v = jnp.where(sel, zeros + n_sc, cnt_v)

    stage[0] = sum_v
    stage[1] = cnt_v
    pltpu.sync_copy(stage.at[pl.ds(0, 2)], shared.at[s, pl.ds(0, 2)])
    plsc.subcore_barrier()

    g0 = b_local * _WPB
    tot_sum = zeros
    tot_cnt = zeros
    for j in range(_WPB):
        pltpu.sync_copy(shared.at[g0 + j, pl.ds(0, 2)], stage.at[pl.ds(2, 2)])
        tot_sum = tot_sum + stage[2]
        tot_cnt = tot_cnt + stage[3]
    mean_v = tot_sum / jnp.maximum(tot_cnt, 1.0)
    means_v[...] = mean_v

    for t in range(_NCHUNK):
        bi = t & 1
        if t + 1 < _NCHUNK:
            pend[(t + 1) & 1] = start_copy(t + 1)
        dx, dg = pend[bi]
        dx.wait()
        dg.wait()

        @plsc.parallel_loop(0, _ROWS, 1)
        def vec2(r):
            for u in range(512 // _L):
                x = xbuf[bi, r, pl.ds(u * _L, _L)]
                g = gbuf[bi, r, pl.ds(u * _L, _L)]
                m = plsc.load_gather(means_v, [g])
                h = jnp.maximum(jnp.abs(x - m) - _M_VAR, 0.0)
                fi = g * _L + lanes + (u & 1) * (_L * _L)
                plsc.addupdate_scatter(pull_acc, [fi], h * h)

    pull_v = zeros
    for cc in range(9):
        p_sc = jnp.sum(pull_acc[pl.ds(cc * _L, _L)] +
                       pull_acc[pl.ds(_L * _L + cc * _L, _L)])
        pull_v = jnp.where(ci == cc, zeros + p_sc, pull_v)
    stage[2] = pull_v
    pltpu.sync_copy(stage.at[pl.ds(2, 1)], shared.at[s, pl.ds(2, 1)])
    plsc.subcore_barrier()

    @pl.when(q == 0)
    def _():
        tot_pull = zeros
        for j in range(_WPB):
            pltpu.sync_copy(shared.at[g0 + j, pl.ds(2, 1)],
                            stage.at[pl.ds(3, 1)])
            tot_pull = tot_pull + stage[3]
        stage[0] = mean_v
        stage[1] = tot_cnt
        stage[2] = tot_pull / jnp.maximum(tot_cnt, 1.0)
        stage[3] = zeros
        pltpu.sync_copy(stage, table_hbm.at[c * _BPC + b_local])


def _final_kernel(table_hbm, out_hbm, tv, ov):
    c = lax.axis_index("c")
    s = lax.axis_index("s")

    @pl.when((c == 0) & (s == 0))
    def _():
        pltpu.sync_copy(table_hbm, tv)
        ci = lax.iota(jnp.int32, _L)
        cls_ok = (ci >= 1) & (ci <= 8)

        maxl = jnp.int32(0)
        for b in range(_B):
            cnt = tv[b, 1]
            maxl = jnp.maximum(maxl, jnp.max(jnp.where(cnt > 0, ci, 0)))

        pull_sum = jnp.float32(0.0)
        pull_cnt = jnp.float32(0.0)
        push_sum = jnp.float32(0.0)
        push_cnt = jnp.float32(0.0)
        for b in range(_B):
            mean = tv[b, 0]
            cnt = tv[b, 1]
            pterm = tv[b, 2]
            present = cls_ok & (cnt > 0.0) & (ci <= maxl)
            presf = jnp.where(present, 1.0, 0.0)
            pull_sum = pull_sum + jnp.sum(jnp.where(present, pterm, 0.0))
            pull_cnt = pull_cnt + jnp.sum(presf)
            for i in range(1, 9):
                sel = ci == i
                mi = jnp.sum(jnp.where(sel, mean, 0.0))
                pi = jnp.sum(jnp.where(sel, presf, 0.0))
                d = jnp.maximum(2.0 * _M_DIST - jnp.abs(mean - mi), 0.0)
                pairf = jnp.where(present & (~sel), pi, 0.0)
                push_sum = push_sum + jnp.sum(d * d * pairf)
                push_cnt = push_cnt + jnp.sum(pairf)

        zv = jnp.zeros((_L,), jnp.float32)
        pull_loss = (zv + pull_sum) / jnp.maximum(zv + pull_cnt, 1.0) * _VAR_W
        push_loss = (zv + push_sum) / jnp.maximum(zv + push_cnt, 1.0) * _DIST_W
        ov[...] = pull_loss + push_loss
        pltpu.sync_copy(ov, out_hbm)


def _build_calls():
    mesh = plsc.VectorSubcoreMesh(core_axis_name="c", subcore_axis_name="s",
                                  num_cores=_NC, num_subcores=_NS)
    params = pltpu.CompilerParams(needs_layout_passes=False)
    phase_params = pltpu.CompilerParams(needs_layout_passes=False,
                                        use_tc_tiling_on_sc=True)
    phase = pl.kernel(
        _phase_kernel,
        out_type=jax.ShapeDtypeStruct((_B, 4, _L), jnp.float32),
        mesh=mesh,
        compiler_params=phase_params,
        scratch_types=[
            pltpu.VMEM((2, _ROWS, 512), jnp.float32),
            pltpu.VMEM((2, _ROWS, 512), jnp.int32),
            pltpu.VMEM((2 * _L * _L,), jnp.float32),
            pltpu.VMEM((2 * _L * _L,), jnp.float32),
            pltpu.VMEM((2 * _L * _L,), jnp.float32),
            pltpu.VMEM((_L,), jnp.float32),
            pltpu.VMEM((4, _L), jnp.float32),
            pltpu.VMEM_SHARED((_NS, 4, _L), jnp.float32),
            pltpu.SemaphoreType.DMA,
            pltpu.SemaphoreType.DMA,
            pltpu.SemaphoreType.DMA,
            pltpu.SemaphoreType.DMA,
        ],
    )
    final = pl.kernel(
        _final_kernel,
        out_type=jax.ShapeDtypeStruct((_L,), jnp.float32),
        mesh=mesh,
        compiler_params=params,
        scratch_types=[
            pltpu.VMEM((_B, 4, _L), jnp.float32),
            pltpu.VMEM((_L,), jnp.float32),
        ],
    )
    return phase, final


def kernel(featmap, gt):
    phase, final = _build_calls()
    feat = featmap.reshape(_B, 512, 512)
    g = gt.reshape(_B, 512, 512).astype(jnp.int32)
    table = phase(feat, g)
    out = final(table)
    return out[0]
